# Initial kernel scaffold; baseline (speedup 1.0000x reference)
import jax
import jax.numpy as jnp
from jax.experimental import pallas as pl


def kernel(data, layer_parameters, wt, bt, W0, b0, W1, b1, W2, b2, W3, b3):
    raise NotImplementedError("write your pallas kernel here")


# speedup vs baseline: 3.3152x; 3.3152x over previous
import jax
import jax.numpy as jnp
from jax.experimental import pallas as pl


def _fused_kernel(data_ref, len_ref, wt_ref, bt_ref,
                  w0_ref, b0_ref, w1_ref, b1_ref,
                  w2_ref, b2_ref, w3_ref, b3_ref, out_ref):
    data = data_ref[...]
    lens = len_ref[...]
    Bc, Tc = data.shape
    t_idx = jax.lax.broadcasted_iota(jnp.int32, (Bc, Tc), 1)
    mask = (t_idx < lens).astype(data.dtype)
    s = jnp.sum(data * mask, axis=1, keepdims=True)
    lenf = lens.astype(data.dtype)
    agg = s * wt_ref[...] + lenf * bt_ref[...]
    h = jnp.maximum(
        jnp.dot(agg, w0_ref[...], preferred_element_type=jnp.float32)
        + b0_ref[...], 0.0)
    h = jnp.maximum(
        jnp.dot(h, w1_ref[...], preferred_element_type=jnp.float32)
        + b1_ref[...], 0.0)
    h = jnp.maximum(
        jnp.dot(h, w2_ref[...], preferred_element_type=jnp.float32)
        + b2_ref[...], 0.0)
    out_ref[...] = (
        jnp.dot(h, w3_ref[...], preferred_element_type=jnp.float32)
        + b3_ref[...])


def kernel(data, layer_parameters, wt, bt, W0, b0, W1, b1, W2, b2, W3, b3):
    B, T = data.shape
    H = wt.shape[0]
    lens2d = layer_parameters.reshape(B, 1)
    return pl.pallas_call(
        _fused_kernel,
        out_shape=jax.ShapeDtypeStruct((B, H), jnp.float32),
    )(data, lens2d, wt.reshape(1, H), bt.reshape(1, H),
      W0, b0.reshape(1, H), W1, b1.reshape(1, H),
      W2, b2.reshape(1, H), W3, b3.reshape(1, H))


# speedup vs baseline: 3.3494x; 1.0103x over previous
import jax
import jax.numpy as jnp
from jax.experimental import pallas as pl
from jax.experimental.pallas import tpu as pltpu


def _fused_kernel(data_ref, len_ref, wt_ref, bt_ref,
                  w0_hbm, b0_ref, w1_hbm, b1_ref,
                  w2_hbm, b2_ref, w3_hbm, b3_ref, out_ref,
                  w0_v, w1_v, w2_v, w3_v,
                  sem0, sem1, sem2, sem3):
    cp0 = pltpu.make_async_copy(w0_hbm, w0_v, sem0)
    cp1 = pltpu.make_async_copy(w1_hbm, w1_v, sem1)
    cp2 = pltpu.make_async_copy(w2_hbm, w2_v, sem2)
    cp3 = pltpu.make_async_copy(w3_hbm, w3_v, sem3)
    cp0.start()
    cp1.start()
    cp2.start()
    cp3.start()

    data = data_ref[...]
    lens = len_ref[...]
    Bc, Tc = data.shape
    t_idx = jax.lax.broadcasted_iota(jnp.int32, (Bc, Tc), 1)
    mask = (t_idx < lens).astype(data.dtype)
    s = jnp.sum(data * mask, axis=1, keepdims=True)
    lenf = lens.astype(data.dtype)
    agg = s * wt_ref[...] + lenf * bt_ref[...]

    cp0.wait()
    h = jnp.maximum(
        jnp.dot(agg, w0_v[...], preferred_element_type=jnp.float32)
        + b0_ref[...], 0.0)
    cp1.wait()
    h = jnp.maximum(
        jnp.dot(h, w1_v[...], preferred_element_type=jnp.float32)
        + b1_ref[...], 0.0)
    cp2.wait()
    h = jnp.maximum(
        jnp.dot(h, w2_v[...], preferred_element_type=jnp.float32)
        + b2_ref[...], 0.0)
    cp3.wait()
    out_ref[...] = (
        jnp.dot(h, w3_v[...], preferred_element_type=jnp.float32)
        + b3_ref[...])


def kernel(data, layer_parameters, wt, bt, W0, b0, W1, b1, W2, b2, W3, b3):
    B, T = data.shape
    H = wt.shape[0]
    lens2d = layer_parameters.reshape(B, 1)
    vmem = pl.BlockSpec(memory_space=pltpu.MemorySpace.VMEM)
    hbm = pl.BlockSpec(memory_space=pl.ANY)
    return pl.pallas_call(
        _fused_kernel,
        out_shape=jax.ShapeDtypeStruct((B, H), jnp.float32),
        in_specs=[vmem, vmem, vmem, vmem,
                  hbm, vmem, hbm, vmem,
                  hbm, vmem, hbm, vmem],
        out_specs=vmem,
        scratch_shapes=[
            pltpu.VMEM((H, H), jnp.float32),
            pltpu.VMEM((H, H), jnp.float32),
            pltpu.VMEM((H, H), jnp.float32),
            pltpu.VMEM((H, H), jnp.float32),
            pltpu.SemaphoreType.DMA,
            pltpu.SemaphoreType.DMA,
            pltpu.SemaphoreType.DMA,
            pltpu.SemaphoreType.DMA,
        ],
    )(data, lens2d, wt.reshape(1, H), bt.reshape(1, H),
      W0, b0.reshape(1, H), W1, b1.reshape(1, H),
      W2, b2.reshape(1, H), W3, b3.reshape(1, H))


# speedup vs baseline: 3.3880x; 1.0115x over previous
import jax
import jax.numpy as jnp
from jax.experimental import pallas as pl
from jax.experimental.pallas import tpu as pltpu

_NCHUNK = 4


def _fused_kernel(data_ref, len_ref, wt_ref, bt_ref,
                  w0_hbm, b0_ref, w1_hbm, b1_ref,
                  w2_hbm, b2_ref, w3_hbm, b3_ref, out_ref,
                  w0_v, w1_v, w2_v, w3_v, sems):
    H = w0_v.shape[0]
    ck = H // _NCHUNK
    copies = []
    for i, (src, dst) in enumerate(((w0_hbm, w0_v), (w1_hbm, w1_v),
                                    (w2_hbm, w2_v), (w3_hbm, w3_v))):
        for j in range(_NCHUNK):
            cp = pltpu.make_async_copy(src.at[pl.ds(j * ck, ck), :],
                                       dst.at[pl.ds(j * ck, ck), :],
                                       sems.at[i * _NCHUNK + j])
            cp.start()
            copies.append(cp)

    data = data_ref[...]
    lens = len_ref[...]
    Bc, Tc = data.shape
    t_idx = jax.lax.broadcasted_iota(jnp.int32, (Bc, Tc), 1)
    mask = (t_idx < lens).astype(data.dtype)
    s = jnp.sum(data * mask, axis=1, keepdims=True)
    lenf = lens.astype(data.dtype)
    h = s * wt_ref[...] + lenf * bt_ref[...]

    for li, (w_v, b_ref) in enumerate(((w0_v, b0_ref), (w1_v, b1_ref),
                                       (w2_v, b2_ref), (w3_v, b3_ref))):
        acc = b_ref[...]
        for j in range(_NCHUNK):
            copies[li * _NCHUNK + j].wait()
            acc = acc + jnp.dot(h[:, j * ck:(j + 1) * ck],
                                w_v[pl.ds(j * ck, ck), :],
                                preferred_element_type=jnp.float32)
        h = jnp.maximum(acc, 0.0) if li < 3 else acc
    out_ref[...] = h


def kernel(data, layer_parameters, wt, bt, W0, b0, W1, b1, W2, b2, W3, b3):
    B, T = data.shape
    H = wt.shape[0]
    lens2d = layer_parameters.reshape(B, 1)
    vmem = pl.BlockSpec(memory_space=pltpu.MemorySpace.VMEM)
    hbm = pl.BlockSpec(memory_space=pl.ANY)
    return pl.pallas_call(
        _fused_kernel,
        out_shape=jax.ShapeDtypeStruct((B, H), jnp.float32),
        in_specs=[vmem, vmem, vmem, vmem,
                  hbm, vmem, hbm, vmem,
                  hbm, vmem, hbm, vmem],
        out_specs=vmem,
        scratch_shapes=[
            pltpu.VMEM((H, H), jnp.float32),
            pltpu.VMEM((H, H), jnp.float32),
            pltpu.VMEM((H, H), jnp.float32),
            pltpu.VMEM((H, H), jnp.float32),
            pltpu.SemaphoreType.DMA((4 * _NCHUNK,)),
        ],
    )(data, lens2d, wt.reshape(1, H), bt.reshape(1, H),
      W0, b0.reshape(1, H), W1, b1.reshape(1, H),
      W2, b2.reshape(1, H), W3, b3.reshape(1, H))
